# baseline (speedup 1.0000x reference)
import functools

import jax
import jax.numpy as jnp
from jax import lax
from jax.experimental import pallas as pl
from jax.experimental.pallas import tpu as pltpu
from jax.experimental.pallas import tpu_sc as plsc

_N = 10000
_E = 320000
_DIN = 128
_DOUT = 256
_HW = 64
_NROWS = 10112
_NT = 16
_CHUNK = 50
_BCH = 40
_NB = 10
_EPT = _NB * _BCH * _CHUNK
_HRPT = _N // _NT
_RPT = _NROWS // _NT


def _make_sc_aggregate():
    mesh = plsc.VectorSubcoreMesh(core_axis_name="c", subcore_axis_name="s")

    @functools.partial(
        pl.kernel,
        mesh=mesh,
        out_type=(
            jax.ShapeDtypeStruct((2, _NROWS, _HW), jnp.float32),
            jax.ShapeDtypeStruct((2, _NROWS), jnp.float32),
        ),
        scratch_types=[
            pltpu.VMEM((_BCH, _CHUNK), jnp.int32),
            pltpu.VMEM((_BCH, _CHUNK), jnp.int32),
            pltpu.VMEM((_BCH, _CHUNK), jnp.int32),
            pltpu.VMEM((_BCH, _CHUNK), jnp.int32),
            pltpu.VMEM((2 * _CHUNK, _HW), jnp.float32),
            pltpu.VMEM((64,), jnp.float32),
            pltpu.VMEM_SHARED((_NROWS, _HW), jnp.float32),
            pltpu.VMEM_SHARED((_NROWS, _HW), jnp.float32),
            pltpu.VMEM_SHARED((_NROWS,), jnp.float32),
            pltpu.SemaphoreType.DMA,
            pltpu.SemaphoreType.DMA,
            pltpu.SemaphoreType.DMA,
            pltpu.SemaphoreType.DMA,
        ],
        compiler_params=pltpu.CompilerParams(use_tc_tiling_on_sc=False),
    )
    def sc_body(hp_hbm, src_hbm, dst_hbm, out_hbm, outd_hbm,
                sidx_a, didx_a, sidx_b, didx_b, rows_v, ones_v,
                h_spm, acc, accd, sem0, sem1, semi, semd):
        c = lax.axis_index("c")
        s = lax.axis_index("s")
        buf0 = rows_v.at[pl.ds(0, _CHUNK)]
        buf1 = rows_v.at[pl.ds(_CHUNK, _CHUNK)]
        row0 = s * _RPT

        hrow0 = s * _HRPT
        pltpu.async_copy(hp_hbm.at[pl.ds(hrow0, _HRPT), pl.ds(c * _HW, _HW)],
                         h_spm.at[pl.ds(hrow0, _HRPT)], semi)
        pltpu.async_copy(src_hbm.at[s, pl.ds(0, _BCH)], sidx_a, sem0)
        pltpu.async_copy(dst_hbm.at[s, pl.ds(0, _BCH)], didx_a, sem0)

        def zrow(i, carry):
            for k in range(_HW // 16):
                rows_v[i, pl.ds(k * 16, 16)] = jnp.zeros((16,), jnp.float32)
            return carry

        lax.fori_loop(0, 2 * _CHUNK, zrow, 0)
        for k in range(4):
            ones_v[pl.ds(k * 16, 16)] = jnp.ones((16,), jnp.float32)
        ones_c = ones_v.at[pl.ds(0, _CHUNK)]

        off = 0
        while off < _RPT:
            n = min(2 * _CHUNK, _RPT - off)
            pltpu.sync_copy(rows_v.at[pl.ds(0, n)], acc.at[pl.ds(row0 + off, n)])
            off += n
        zrow64 = rows_v.at[0]
        for doff in (0, 64, 128, 192, 256, 320, 384, 448, 512, 568):
            pltpu.sync_copy(zrow64, accd.at[pl.ds(row0 + doff, 64)])

        pltpu.make_async_copy(hp_hbm.at[pl.ds(hrow0, _HRPT), pl.ds(0, _HW)],
                              h_spm.at[pl.ds(hrow0, _HRPT)], semi).wait()
        plsc.subcore_barrier()

        def gather(sidx, j, buf, sem):
            pltpu.async_copy(h_spm.at[sidx.at[j]], buf, sem)

        def drain(buf, sem):
            pltpu.make_async_copy(h_spm.at[pl.ds(0, _CHUNK)], buf, sem).wait()

        def scat(didx, j, buf, dcond):
            @pl.when(dcond)
            def _():
                pltpu.async_copy(ones_c, accd.at[didx.at[j]], semd, add=True)
            pltpu.sync_copy(buf, acc.at[didx.at[j]], add=True)

        idx_bufs = [(sidx_a, didx_a), (sidx_b, didx_b)]
        for b in range(_NB):
            dcond = (c == 0) if b < _NB // 2 else (c == 1)
            sidx, didx = idx_bufs[b % 2]
            sidx_n, didx_n = idx_bufs[(b + 1) % 2]
            pltpu.make_async_copy(src_hbm.at[s, pl.ds(0, _BCH)], sidx,
                                  sem0).wait()
            pltpu.make_async_copy(dst_hbm.at[s, pl.ds(0, _BCH)], didx,
                                  sem0).wait()
            if b + 1 < _NB:
                nb0 = (b + 1) * _BCH
                pltpu.async_copy(src_hbm.at[s, pl.ds(nb0, _BCH)], sidx_n, sem0)
                pltpu.async_copy(dst_hbm.at[s, pl.ds(nb0, _BCH)], didx_n, sem0)

            gather(sidx, 0, buf0, sem1)

            def pair(jj, carry):
                j = jj * 2
                gather(sidx, j + 1, buf1, semi)
                drain(buf0, sem1)
                scat(didx, j, buf0, dcond)
                gather(sidx, j + 2, buf0, sem1)
                drain(buf1, semi)
                scat(didx, j + 1, buf1, dcond)
                return carry

            lax.fori_loop(0, _BCH // 2 - 1, pair, 0)
            gather(sidx, _BCH - 1, buf1, semi)
            drain(buf0, sem1)
            scat(didx, _BCH - 2, buf0, dcond)
            drain(buf1, semi)
            scat(didx, _BCH - 1, buf1, dcond)

            @pl.when(dcond)
            def _():
                def ddrain(i, carry):
                    pltpu.make_async_copy(ones_c, accd.at[pl.ds(0, _CHUNK)],
                                          semd).wait()
                    return carry

                lax.fori_loop(0, _BCH, ddrain, 0)

        plsc.subcore_barrier()

        pltpu.sync_copy(acc.at[pl.ds(row0, _RPT)], out_hbm.at[c, pl.ds(row0, _RPT)])
        pltpu.sync_copy(accd.at[pl.ds(row0, _RPT)], outd_hbm.at[c, pl.ds(row0, _RPT)])

    return sc_body


_sc_aggregate = _make_sc_aggregate()


def _finish_body(p0_ref, p1_ref, deg_ref, h_ref, g_ref, b_ref, o_ref):
    msg = jnp.concatenate([p0_ref[0], p1_ref[0]], axis=1)
    deg = deg_ref[0, 0][:, None]
    safe = jnp.maximum(deg, 1.0)
    inv2 = jnp.where(deg > 0, 1.0 / (safe * safe), 0.0)
    ahn = msg * inv2
    hb = h_ref[...]
    hc = jnp.concatenate([hb, ahn], axis=1)
    mu = jnp.mean(hc, axis=1, keepdims=True)
    d = hc - mu
    var = jnp.mean(d * d, axis=1, keepdims=True)
    o_ref[...] = d * lax.rsqrt(var + 1e-5) * g_ref[...] + b_ref[...]


def _finish(partials, deg3, h, gamma2, beta2):
    R = 2000
    grid = (_N // R,)
    return pl.pallas_call(
        _finish_body,
        grid=grid,
        in_specs=[
            pl.BlockSpec((1, R, _HW), lambda i: (0, i, 0)),
            pl.BlockSpec((1, R, _HW), lambda i: (1, i, 0)),
            pl.BlockSpec((1, 1, R), lambda i: (i, 0, 0)),
            pl.BlockSpec((R, _DIN), lambda i: (i, 0)),
            pl.BlockSpec((1, _DOUT), lambda i: (0, 0)),
            pl.BlockSpec((1, _DOUT), lambda i: (0, 0)),
        ],
        out_specs=pl.BlockSpec((R, _DOUT), lambda i: (i, 0)),
        out_shape=jax.ShapeDtypeStruct((_N, _DOUT), jnp.float32),
    )(partials, partials, deg3, h, gamma2, beta2)


def kernel(h, edge_index, ln_gamma, ln_beta):
    src = edge_index[0]
    dst = edge_index[1]

    srcp = src.reshape(_NT, _NB * _BCH, _CHUNK)
    dstp = dst.reshape(_NT, _NB * _BCH, _CHUNK)

    partials, pdeg = _sc_aggregate(h, srcp, dstp)
    deg3 = (pdeg[0] + pdeg[1])[: _N].reshape(_N // 2000, 1, 2000)
    return _finish(partials, deg3, h,
                   ln_gamma.reshape(1, _DOUT), ln_beta.reshape(1, _DOUT))


# speedup vs baseline: 1.0907x; 1.0907x over previous
import functools

import jax
import jax.numpy as jnp
from jax import lax
from jax.experimental import pallas as pl
from jax.experimental.pallas import tpu as pltpu
from jax.experimental.pallas import tpu_sc as plsc

_N = 10000
_E = 320000
_DIN = 128
_DOUT = 256
_HW = 64
_NROWS = 10112
_NT = 16
_CHUNK = 48
_BCH = 60
_NB = 7
_EPT = _NB * _BCH * _CHUNK
_HRPT = _N // _NT
_RPT = _NROWS // _NT


def _make_sc_aggregate():
    mesh = plsc.VectorSubcoreMesh(core_axis_name="c", subcore_axis_name="s")

    @functools.partial(
        pl.kernel,
        mesh=mesh,
        out_type=(
            jax.ShapeDtypeStruct((2, _NROWS, _HW), jnp.float32),
            jax.ShapeDtypeStruct((2, _NROWS), jnp.float32),
        ),
        scratch_types=[
            pltpu.VMEM((_BCH, _CHUNK), jnp.int32),
            pltpu.VMEM((_BCH, _CHUNK), jnp.int32),
            pltpu.VMEM((_BCH, _CHUNK), jnp.int32),
            pltpu.VMEM((_BCH, _CHUNK), jnp.int32),
            pltpu.VMEM((2 * _CHUNK, _HW), jnp.float32),
            pltpu.VMEM((64,), jnp.float32),
            pltpu.VMEM_SHARED((_NROWS, _HW), jnp.float32),
            pltpu.VMEM_SHARED((_NROWS, _HW), jnp.float32),
            pltpu.VMEM_SHARED((_NROWS,), jnp.float32),
            pltpu.SemaphoreType.DMA,
            pltpu.SemaphoreType.DMA,
            pltpu.SemaphoreType.DMA,
            pltpu.SemaphoreType.DMA,
        ],
        compiler_params=pltpu.CompilerParams(use_tc_tiling_on_sc=False),
    )
    def sc_body(hp_hbm, src_hbm, dst_hbm, out_hbm, outd_hbm,
                sidx_a, didx_a, sidx_b, didx_b, rows_v, ones_v,
                h_spm, acc, accd, sem0, sem1, semi, semd):
        c = lax.axis_index("c")
        s = lax.axis_index("s")
        buf0 = rows_v.at[pl.ds(0, _CHUNK)]
        buf1 = rows_v.at[pl.ds(_CHUNK, _CHUNK)]
        row0 = s * _RPT

        hrow0 = s * _HRPT
        pltpu.async_copy(hp_hbm.at[pl.ds(hrow0, _HRPT), pl.ds(c * _HW, _HW)],
                         h_spm.at[pl.ds(hrow0, _HRPT)], semi)
        pltpu.async_copy(src_hbm.at[s, pl.ds(0, _BCH)], sidx_a, sem0)
        pltpu.async_copy(dst_hbm.at[s, pl.ds(0, _BCH)], didx_a, sem0)

        def zrow(i, carry):
            for k in range(_HW // 16):
                rows_v[i, pl.ds(k * 16, 16)] = jnp.zeros((16,), jnp.float32)
            return carry

        lax.fori_loop(0, 2 * _CHUNK, zrow, 0)
        for k in range(4):
            ones_v[pl.ds(k * 16, 16)] = jnp.ones((16,), jnp.float32)
        ones_c = ones_v.at[pl.ds(0, _CHUNK)]

        off = 0
        while off < _RPT:
            n = min(2 * _CHUNK, _RPT - off)
            pltpu.sync_copy(rows_v.at[pl.ds(0, n)], acc.at[pl.ds(row0 + off, n)])
            off += n
        zrow64 = rows_v.at[0]
        for doff in (0, 64, 128, 192, 256, 320, 384, 448, 512, 568):
            pltpu.sync_copy(zrow64, accd.at[pl.ds(row0 + doff, 64)])

        pltpu.make_async_copy(hp_hbm.at[pl.ds(hrow0, _HRPT), pl.ds(0, _HW)],
                              h_spm.at[pl.ds(hrow0, _HRPT)], semi).wait()
        plsc.subcore_barrier()

        def gather(sidx, j, buf, sem):
            pltpu.async_copy(h_spm.at[sidx.at[j]], buf, sem)

        def drain(buf, sem):
            pltpu.make_async_copy(h_spm.at[pl.ds(0, _CHUNK)], buf, sem).wait()

        def scat(didx, j, buf, dcond):
            @pl.when(dcond)
            def _():
                pltpu.async_copy(ones_c, accd.at[didx.at[j]], semd, add=True)
            pltpu.sync_copy(buf, acc.at[didx.at[j]], add=True)

        idx_bufs = [(sidx_a, didx_a), (sidx_b, didx_b)]
        for b in range(_NB):
            dcond = (c == 0) if b < _NB // 2 else (c == 1)
            sidx, didx = idx_bufs[b % 2]
            sidx_n, didx_n = idx_bufs[(b + 1) % 2]
            pltpu.make_async_copy(src_hbm.at[s, pl.ds(0, _BCH)], sidx,
                                  sem0).wait()
            pltpu.make_async_copy(dst_hbm.at[s, pl.ds(0, _BCH)], didx,
                                  sem0).wait()
            if b + 1 < _NB:
                nb0 = (b + 1) * _BCH
                pltpu.async_copy(src_hbm.at[s, pl.ds(nb0, _BCH)], sidx_n, sem0)
                pltpu.async_copy(dst_hbm.at[s, pl.ds(nb0, _BCH)], didx_n, sem0)

            gather(sidx, 0, buf0, sem1)

            def pair(jj, carry):
                j = jj * 2
                gather(sidx, j + 1, buf1, semi)
                drain(buf0, sem1)
                scat(didx, j, buf0, dcond)
                gather(sidx, j + 2, buf0, sem1)
                drain(buf1, semi)
                scat(didx, j + 1, buf1, dcond)
                return carry

            lax.fori_loop(0, _BCH // 2 - 1, pair, 0)
            gather(sidx, _BCH - 1, buf1, semi)
            drain(buf0, sem1)
            scat(didx, _BCH - 2, buf0, dcond)
            drain(buf1, semi)
            scat(didx, _BCH - 1, buf1, dcond)

            @pl.when(dcond)
            def _():
                def ddrain(i, carry):
                    pltpu.make_async_copy(ones_c, accd.at[pl.ds(0, _CHUNK)],
                                          semd).wait()
                    return carry

                lax.fori_loop(0, _BCH, ddrain, 0)

        plsc.subcore_barrier()

        pltpu.sync_copy(acc.at[pl.ds(row0, _RPT)], out_hbm.at[c, pl.ds(row0, _RPT)])
        pltpu.sync_copy(accd.at[pl.ds(row0, _RPT)], outd_hbm.at[c, pl.ds(row0, _RPT)])

    return sc_body


_sc_aggregate = _make_sc_aggregate()


def _finish_body(p0_ref, p1_ref, deg_ref, h_ref, g_ref, b_ref, o_ref):
    msg = jnp.concatenate([p0_ref[0], p1_ref[0]], axis=1)
    deg = deg_ref[0, 0][:, None]
    safe = jnp.maximum(deg, 1.0)
    inv2 = jnp.where(deg > 0, 1.0 / (safe * safe), 0.0)
    ahn = msg * inv2
    hb = h_ref[...]
    hc = jnp.concatenate([hb, ahn], axis=1)
    mu = jnp.mean(hc, axis=1, keepdims=True)
    d = hc - mu
    var = jnp.mean(d * d, axis=1, keepdims=True)
    o_ref[...] = d * lax.rsqrt(var + 1e-5) * g_ref[...] + b_ref[...]


def _finish(partials, deg3, h, gamma2, beta2):
    R = 2000
    grid = (_N // R,)
    return pl.pallas_call(
        _finish_body,
        grid=grid,
        in_specs=[
            pl.BlockSpec((1, R, _HW), lambda i: (0, i, 0)),
            pl.BlockSpec((1, R, _HW), lambda i: (1, i, 0)),
            pl.BlockSpec((1, 1, R), lambda i: (i, 0, 0)),
            pl.BlockSpec((R, _DIN), lambda i: (i, 0)),
            pl.BlockSpec((1, _DOUT), lambda i: (0, 0)),
            pl.BlockSpec((1, _DOUT), lambda i: (0, 0)),
        ],
        out_specs=pl.BlockSpec((R, _DOUT), lambda i: (i, 0)),
        out_shape=jax.ShapeDtypeStruct((_N, _DOUT), jnp.float32),
    )(partials, partials, deg3, h, gamma2, beta2)


def kernel(h, edge_index, ln_gamma, ln_beta):
    src = edge_index[0]
    dst = edge_index[1]

    ept_real = _E // _NT
    fill = jnp.full((_NT, _EPT - ept_real), _N, jnp.int32)
    srcp = jnp.concatenate([src.reshape(_NT, ept_real), fill],
                           axis=1).reshape(_NT, _NB * _BCH, _CHUNK)
    dstp = jnp.concatenate([dst.reshape(_NT, ept_real), fill],
                           axis=1).reshape(_NT, _NB * _BCH, _CHUNK)

    partials, pdeg = _sc_aggregate(h, srcp, dstp)
    deg3 = (pdeg[0] + pdeg[1])[: _N].reshape(_N // 2000, 1, 2000)
    return _finish(partials, deg3, h,
                   ln_gamma.reshape(1, _DOUT), ln_beta.reshape(1, _DOUT))


# speedup vs baseline: 1.1192x; 1.0261x over previous
import functools

import jax
import jax.numpy as jnp
from jax import lax
from jax.experimental import pallas as pl
from jax.experimental.pallas import tpu as pltpu
from jax.experimental.pallas import tpu_sc as plsc

_N = 10000
_E = 320000
_DIN = 128
_DOUT = 256
_HW = 64
_NROWS = 10112
_NT = 16
_CHUNK = 48
_BCH = 60
_NB = 7
_EPT = _NB * _BCH * _CHUNK
_HRPT = _N // _NT
_RPT = _NROWS // _NT


def _make_sc_aggregate():
    mesh = plsc.VectorSubcoreMesh(core_axis_name="c", subcore_axis_name="s")

    @functools.partial(
        pl.kernel,
        mesh=mesh,
        out_type=(
            jax.ShapeDtypeStruct((2, _NROWS, _HW), jnp.float32),
            jax.ShapeDtypeStruct((2, _NROWS), jnp.float32),
        ),
        scratch_types=[
            pltpu.VMEM((_BCH, _CHUNK), jnp.int32),
            pltpu.VMEM((_BCH, _CHUNK), jnp.int32),
            pltpu.VMEM((_BCH, _CHUNK), jnp.int32),
            pltpu.VMEM((_BCH, _CHUNK), jnp.int32),
            pltpu.VMEM((4 * _CHUNK, _HW), jnp.float32),
            pltpu.VMEM((64,), jnp.float32),
            pltpu.VMEM_SHARED((_NROWS, _HW), jnp.float32),
            pltpu.VMEM_SHARED((_NROWS, _HW), jnp.float32),
            pltpu.VMEM_SHARED((_NROWS,), jnp.float32),
            pltpu.SemaphoreType.DMA,
            pltpu.SemaphoreType.DMA,
            pltpu.SemaphoreType.DMA,
            pltpu.SemaphoreType.DMA,
            pltpu.SemaphoreType.DMA,
            pltpu.SemaphoreType.DMA,
            pltpu.SemaphoreType.DMA,
            pltpu.SemaphoreType.DMA,
            pltpu.SemaphoreType.DMA,
            pltpu.SemaphoreType.DMA,
            pltpu.SemaphoreType.DMA,
            pltpu.SemaphoreType.DMA,
        ],
        compiler_params=pltpu.CompilerParams(use_tc_tiling_on_sc=False),
    )
    def sc_body(hp_hbm, src_hbm, dst_hbm, out_hbm, outd_hbm,
                sidx_a, didx_a, sidx_b, didx_b, rows_v, ones_v,
                h_spm, acc, accd, sem0, sem1, semi, semd,
                g0, g1, g2, g3, s0, s1, s2, s3):
        c = lax.axis_index("c")
        s = lax.axis_index("s")
        bufs = [rows_v.at[pl.ds(q * _CHUNK, _CHUNK)] for q in range(4)]
        gsems = [g0, g1, g2, g3]
        ssems = [s0, s1, s2, s3]
        row0 = s * _RPT

        hrow0 = s * _HRPT
        pltpu.async_copy(hp_hbm.at[pl.ds(hrow0, _HRPT), pl.ds(c * _HW, _HW)],
                         h_spm.at[pl.ds(hrow0, _HRPT)], semi)
        pltpu.async_copy(src_hbm.at[s, pl.ds(0, _BCH)], sidx_a, sem0)
        pltpu.async_copy(dst_hbm.at[s, pl.ds(0, _BCH)], didx_a, sem0)

        def zrow(i, carry):
            for k in range(_HW // 16):
                rows_v[i, pl.ds(k * 16, 16)] = jnp.zeros((16,), jnp.float32)
            return carry

        lax.fori_loop(0, 4 * _CHUNK, zrow, 0)
        for k in range(4):
            ones_v[pl.ds(k * 16, 16)] = jnp.ones((16,), jnp.float32)
        ones_c = ones_v.at[pl.ds(0, _CHUNK)]

        off = 0
        while off < _RPT:
            n = min(4 * _CHUNK, _RPT - off)
            pltpu.sync_copy(rows_v.at[pl.ds(0, n)], acc.at[pl.ds(row0 + off, n)])
            off += n
        zrow64 = rows_v.at[0]
        for doff in (0, 64, 128, 192, 256, 320, 384, 448, 512, 568):
            pltpu.sync_copy(zrow64, accd.at[pl.ds(row0 + doff, 64)])

        pltpu.make_async_copy(hp_hbm.at[pl.ds(hrow0, _HRPT), pl.ds(0, _HW)],
                              h_spm.at[pl.ds(hrow0, _HRPT)], semi).wait()
        plsc.subcore_barrier()

        def gather(sidx, j, q):
            pltpu.async_copy(h_spm.at[sidx.at[j]], bufs[q], gsems[q])

        def gwait(q):
            pltpu.make_async_copy(h_spm.at[pl.ds(0, _CHUNK)], bufs[q],
                                  gsems[q]).wait()

        def scat(didx, j, q, dcond):
            @pl.when(dcond)
            def _():
                pltpu.async_copy(ones_c, accd.at[didx.at[j]], semd, add=True)
            pltpu.async_copy(bufs[q], acc.at[didx.at[j]], ssems[q], add=True)

        def swait(q):
            pltpu.make_async_copy(bufs[q], acc.at[pl.ds(0, _CHUNK)],
                                  ssems[q]).wait()

        idx_bufs = [(sidx_a, didx_a), (sidx_b, didx_b)]
        for b in range(_NB):
            dcond = (c == 0) if b < _NB // 2 else (c == 1)
            sidx, didx = idx_bufs[b % 2]
            sidx_n, didx_n = idx_bufs[(b + 1) % 2]
            pltpu.make_async_copy(src_hbm.at[s, pl.ds(0, _BCH)], sidx,
                                  sem0).wait()
            pltpu.make_async_copy(dst_hbm.at[s, pl.ds(0, _BCH)], didx,
                                  sem0).wait()
            if b + 1 < _NB:
                nb0 = (b + 1) * _BCH
                pltpu.async_copy(src_hbm.at[s, pl.ds(nb0, _BCH)], sidx_n, sem0)
                pltpu.async_copy(dst_hbm.at[s, pl.ds(nb0, _BCH)], didx_n, sem0)

            gather(sidx, 0, 0)
            gather(sidx, 1, 1)

            def quad(g, carry):
                j0 = g * 4
                gwait(0)
                scat(didx, j0, 0, dcond)

                @pl.when(g > 0)
                def _():
                    swait(2)
                gather(sidx, j0 + 2, 2)
                gwait(1)
                scat(didx, j0 + 1, 1, dcond)

                @pl.when(g > 0)
                def _():
                    swait(3)
                gather(sidx, j0 + 3, 3)
                gwait(2)
                scat(didx, j0 + 2, 2, dcond)
                swait(0)
                gather(sidx, j0 + 4, 0)
                gwait(3)
                scat(didx, j0 + 3, 3, dcond)
                swait(1)
                gather(sidx, j0 + 5, 1)
                return carry

            lax.fori_loop(0, (_BCH - 4) // 4, quad, 0)
            j0 = _BCH - 4
            gwait(0)
            scat(didx, j0, 0, dcond)
            swait(2)
            gather(sidx, j0 + 2, 2)
            gwait(1)
            scat(didx, j0 + 1, 1, dcond)
            swait(3)
            gather(sidx, j0 + 3, 3)
            gwait(2)
            scat(didx, j0 + 2, 2, dcond)
            gwait(3)
            scat(didx, j0 + 3, 3, dcond)
            swait(0)
            swait(1)
            swait(2)
            swait(3)

            @pl.when(dcond)
            def _():
                def ddrain(i, carry):
                    pltpu.make_async_copy(ones_c, accd.at[pl.ds(0, _CHUNK)],
                                          semd).wait()
                    return carry

                lax.fori_loop(0, _BCH, ddrain, 0)

        plsc.subcore_barrier()

        pltpu.sync_copy(acc.at[pl.ds(row0, _RPT)], out_hbm.at[c, pl.ds(row0, _RPT)])
        pltpu.sync_copy(accd.at[pl.ds(row0, _RPT)], outd_hbm.at[c, pl.ds(row0, _RPT)])

    return sc_body


_sc_aggregate = _make_sc_aggregate()


def _finish_body(p0_ref, p1_ref, deg_ref, h_ref, g_ref, b_ref, o_ref):
    msg = jnp.concatenate([p0_ref[0], p1_ref[0]], axis=1)
    deg = deg_ref[0, 0][:, None]
    safe = jnp.maximum(deg, 1.0)
    inv2 = jnp.where(deg > 0, 1.0 / (safe * safe), 0.0)
    ahn = msg * inv2
    hb = h_ref[...]
    hc = jnp.concatenate([hb, ahn], axis=1)
    mu = jnp.mean(hc, axis=1, keepdims=True)
    d = hc - mu
    var = jnp.mean(d * d, axis=1, keepdims=True)
    o_ref[...] = d * lax.rsqrt(var + 1e-5) * g_ref[...] + b_ref[...]


def _finish(partials, deg3, h, gamma2, beta2):
    R = 2000
    grid = (_N // R,)
    return pl.pallas_call(
        _finish_body,
        grid=grid,
        in_specs=[
            pl.BlockSpec((1, R, _HW), lambda i: (0, i, 0)),
            pl.BlockSpec((1, R, _HW), lambda i: (1, i, 0)),
            pl.BlockSpec((1, 1, R), lambda i: (i, 0, 0)),
            pl.BlockSpec((R, _DIN), lambda i: (i, 0)),
            pl.BlockSpec((1, _DOUT), lambda i: (0, 0)),
            pl.BlockSpec((1, _DOUT), lambda i: (0, 0)),
        ],
        out_specs=pl.BlockSpec((R, _DOUT), lambda i: (i, 0)),
        out_shape=jax.ShapeDtypeStruct((_N, _DOUT), jnp.float32),
    )(partials, partials, deg3, h, gamma2, beta2)


def kernel(h, edge_index, ln_gamma, ln_beta):
    src = edge_index[0]
    dst = edge_index[1]

    ept_real = _E // _NT
    fill = jnp.full((_NT, _EPT - ept_real), _N, jnp.int32)
    srcp = jnp.concatenate([src.reshape(_NT, ept_real), fill],
                           axis=1).reshape(_NT, _NB * _BCH, _CHUNK)
    dstp = jnp.concatenate([dst.reshape(_NT, ept_real), fill],
                           axis=1).reshape(_NT, _NB * _BCH, _CHUNK)

    partials, pdeg = _sc_aggregate(h, srcp, dstp)
    deg3 = (pdeg[0] + pdeg[1])[: _N].reshape(_N // 2000, 1, 2000)
    return _finish(partials, deg3, h,
                   ln_gamma.reshape(1, _DOUT), ln_beta.reshape(1, _DOUT))
